# baseline (speedup 1.0000x reference)
import functools

import jax
import jax.numpy as jnp
from jax import lax
from jax.experimental import pallas as pl
from jax.experimental.pallas import tpu as pltpu
from jax.experimental.pallas import tpu_sc as plsc

BATCH = 16384
DIM = 32
VOCAB = 1000000
NW = 32
PER_W = BATCH // NW
CHUNK = 128
NCHUNK = PER_W // CHUNK
NTILE = VOCAB // 128
SROWS = (NTILE + 1) * 32
GT = 12
NGRP = NTILE // GT
KMAX = (NGRP + NW - 1) // NW


def _retile_body(utab, itab, utail, itail, uout, iout, buf0, buf1, sem):
    w = lax.axis_index("s") * 2 + lax.axis_index("c")
    bufs = (buf0, buf1)

    def run(src, dst):
        def fire(k, buf):
            t = w + k * NW
            pltpu.async_copy(
                src.at[:, pl.ds(pl.multiple_of(t * (GT * 128), 128), GT * 128)],
                buf, sem)

        def write(k, buf):
            t = w + k * NW
            pltpu.make_async_copy(
                src.at[:, pl.ds(0, GT * 128)], buf, sem).wait()
            for j in range(GT):
                pltpu.sync_copy(
                    buf.at[:, pl.ds(j * 128, 128)],
                    dst.at[pl.ds((t * GT + j) * 32, 32), :])

        fire(0, buf0)

        def body(k, _):
            @pl.when(w + k * NW < NGRP)
            def _fire():
                b = lax.rem(k, 2)

                @pl.when(b == 0)
                def _():
                    fire(k, buf0)

                @pl.when(b == 1)
                def _():
                    fire(k, buf1)

            bprev = lax.rem(k - 1, 2)

            @pl.when(bprev == 0)
            def _():
                write(k - 1, buf0)

            @pl.when(bprev == 1)
            def _():
                write(k - 1, buf1)

            return 0

        lax.fori_loop(1, KMAX, body, 0)

        @pl.when(w + (KMAX - 1) * NW < NGRP)
        def _():
            blast = lax.rem(KMAX - 1, 2)

            @pl.when(blast == 0)
            def _():
                write(KMAX - 1, buf0)

            @pl.when(blast == 1)
            def _():
                write(KMAX - 1, buf1)

    run(utab, uout)
    run(itab, iout)

    @pl.when(w == 0)
    def _():
        pltpu.sync_copy(utail, buf0.at[:, pl.ds(0, 128)])
        pltpu.sync_copy(buf0.at[:, pl.ds(0, 128)],
                        uout.at[pl.ds(NTILE * 32, 32), :])
        pltpu.sync_copy(itail, buf1.at[:, pl.ds(0, 128)])
        pltpu.sync_copy(buf1.at[:, pl.ds(0, 128)],
                        iout.at[pl.ds(NTILE * 32, 32), :])


_retile = functools.partial(
    pl.kernel,
    mesh=plsc.VectorSubcoreMesh(core_axis_name="c", subcore_axis_name="s"),
    out_type=(jax.ShapeDtypeStruct((SROWS, 128), jnp.float32),
              jax.ShapeDtypeStruct((SROWS, 128), jnp.float32)),
    compiler_params=pltpu.CompilerParams(needs_layout_passes=False),
    scratch_types=[
        pltpu.VMEM((DIM, GT * 128), jnp.float32),
        pltpu.VMEM((DIM, GT * 128), jnp.float32),
        pltpu.SemaphoreType.DMA,
    ],
)(_retile_body)


def _score_body(uid_hbm, pid_hbm, nid_hbm, uflat, iflat, btab, out_hbm,
                uidx_v, pidx_v, nidx_v, ufi, pfi, nfi,
                urows, prows, nrows, bp, bn, outp, outn, sem):
    w = lax.axis_index("s") * 2 + lax.axis_index("c")
    base = w * PER_W

    pltpu.sync_copy(uid_hbm.at[pl.ds(base, PER_W)], uidx_v)
    pltpu.sync_copy(pid_hbm.at[pl.ds(base, PER_W)], pidx_v)
    pltpu.sync_copy(nid_hbm.at[pl.ds(base, PER_W)], nidx_v)

    bias_copies = []
    for c in range(NCHUNK):
        sl = pl.ds(c * CHUNK, CHUNK)
        bias_copies.append(pltpu.async_copy(btab.at[pidx_v.at[sl]], bp.at[sl], sem))
        bias_copies.append(pltpu.async_copy(btab.at[nidx_v.at[sl]], bn.at[sl], sem))
    def fill_base(c, idx_v, fi):
        for k in range(CHUNK // 16):
            r = idx_v[pl.ds(c * CHUNK + k * 16, 16)]
            b = lax.shift_left(lax.shift_right_logical(r, 7), 12) + \
                lax.bitwise_and(r, 127)
            for d in range(DIM):
                fi[pl.ds((c * DIM + d) * CHUNK + k * 16, 16)] = b + d * 128

    for c in range(NCHUNK):
        fill_base(c, uidx_v, ufi)
        fill_base(c, pidx_v, pfi)
        fill_base(c, nidx_v, nfi)
        for d in range(DIM):
            isl = pl.ds((c * DIM + d) * CHUNK, CHUNK)
            dst = pl.ds(d * PER_W + c * CHUNK, CHUNK)
            pltpu.async_copy(uflat.at[ufi.at[isl]], urows.at[dst], sem)
            pltpu.async_copy(iflat.at[pfi.at[isl]], prows.at[dst], sem)
            pltpu.async_copy(iflat.at[nfi.at[isl]], nrows.at[dst], sem)
    for c in bias_copies:
        c.wait()
    pltpu.make_async_copy(uflat.at[pl.ds(0, DIM * PER_W)], urows, sem).wait()
    pltpu.make_async_copy(uflat.at[pl.ds(0, DIM * PER_W)], prows, sem).wait()
    pltpu.make_async_copy(uflat.at[pl.ds(0, DIM * PER_W)], nrows, sem).wait()

    def group(g, _):
        gbase = g * 16
        accp = bp[pl.ds(gbase, 16)]
        accn = bn[pl.ds(gbase, 16)]
        for d in range(DIM):
            uv = urows[pl.ds(d * PER_W + gbase, 16)]
            pv = prows[pl.ds(d * PER_W + gbase, 16)]
            nv = nrows[pl.ds(d * PER_W + gbase, 16)]
            accp = accp + uv * pv
            accn = accn + uv * nv
        outp[pl.ds(gbase, 16)] = accp
        outn[pl.ds(gbase, 16)] = accn
        return 0

    lax.fori_loop(0, PER_W // 16, group, 0)

    pltpu.sync_copy(outp, out_hbm.at[pl.ds(base, PER_W)])
    pltpu.sync_copy(outn, out_hbm.at[pl.ds(base + BATCH, PER_W)])


_score = functools.partial(
    pl.kernel,
    mesh=plsc.VectorSubcoreMesh(core_axis_name="c", subcore_axis_name="s"),
    out_type=jax.ShapeDtypeStruct((2 * BATCH,), jnp.float32),
    compiler_params=pltpu.CompilerParams(needs_layout_passes=False),
    scratch_types=[
        pltpu.VMEM((PER_W,), jnp.int32),
        pltpu.VMEM((PER_W,), jnp.int32),
        pltpu.VMEM((PER_W,), jnp.int32),
        pltpu.VMEM((DIM * PER_W,), jnp.int32),
        pltpu.VMEM((DIM * PER_W,), jnp.int32),
        pltpu.VMEM((DIM * PER_W,), jnp.int32),
        pltpu.VMEM((DIM * PER_W,), jnp.float32),
        pltpu.VMEM((DIM * PER_W,), jnp.float32),
        pltpu.VMEM((DIM * PER_W,), jnp.float32),
        pltpu.VMEM((PER_W,), jnp.float32),
        pltpu.VMEM((PER_W,), jnp.float32),
        pltpu.VMEM((PER_W,), jnp.float32),
        pltpu.VMEM((PER_W,), jnp.float32),
        pltpu.SemaphoreType.DMA,
    ],
)(_score_body)


def kernel(batch_data, u_embedding, i_embedding, i_bias):
    idx = batch_data.astype(jnp.int32)
    tail = jnp.zeros((128, DIM), jnp.float32)
    utail = tail.at[:VOCAB - NTILE * 128].set(u_embedding[NTILE * 128:]).T
    itail = tail.at[:VOCAB - NTILE * 128].set(i_embedding[NTILE * 128:]).T
    ulin, ilin = _retile(u_embedding.T, i_embedding.T, utail, itail)
    logits = _score(idx[:, 0], idx[:, 1], idx[:, 2],
                    ulin.reshape(SROWS * 128), ilin.reshape(SROWS * 128),
                    i_bias.reshape(-1))
    labels = jnp.concatenate([
        jnp.ones((BATCH,), dtype=jnp.float32),
        jnp.zeros((BATCH,), dtype=jnp.float32),
    ])
    return (logits.reshape(2 * BATCH, 1), labels)


# speedup vs baseline: 1.0079x; 1.0079x over previous
import functools

import jax
import jax.numpy as jnp
from jax import lax
from jax.experimental import pallas as pl
from jax.experimental.pallas import tpu as pltpu
from jax.experimental.pallas import tpu_sc as plsc

BATCH = 16384
DIM = 32
VOCAB = 1000000
NW = 32
PER_W = BATCH // NW
CHUNK = 128
NCHUNK = PER_W // CHUNK
NTILE = VOCAB // 128
SROWS = (NTILE + 1) * 32
GT = 8
NGRP = NTILE // GT
KMAX = (NGRP + NW - 1) // NW


def _retile_body(utab, itab, utail, itail, uout, iout, buf0, buf1, sem):
    w = lax.axis_index("s") * 2 + lax.axis_index("c")
    bufs = (buf0, buf1)

    def run(src, dst):
        def fire(k, buf):
            t = w + k * NW
            pltpu.async_copy(
                src.at[:, pl.ds(pl.multiple_of(t * (GT * 128), 128), GT * 128)],
                buf, sem)

        def write(k, buf):
            t = w + k * NW
            pltpu.make_async_copy(
                src.at[:, pl.ds(0, GT * 128)], buf, sem).wait()
            for j in range(GT):
                pltpu.sync_copy(
                    buf.at[:, pl.ds(j * 128, 128)],
                    dst.at[pl.ds((t * GT + j) * 32, 32), :])

        fire(0, buf0)

        def body(k, _):
            @pl.when(w + k * NW < NGRP)
            def _fire():
                b = lax.rem(k, 2)

                @pl.when(b == 0)
                def _():
                    fire(k, buf0)

                @pl.when(b == 1)
                def _():
                    fire(k, buf1)

            bprev = lax.rem(k - 1, 2)

            @pl.when(bprev == 0)
            def _():
                write(k - 1, buf0)

            @pl.when(bprev == 1)
            def _():
                write(k - 1, buf1)

            return 0

        lax.fori_loop(1, KMAX, body, 0)

        @pl.when(w + (KMAX - 1) * NW < NGRP)
        def _():
            blast = lax.rem(KMAX - 1, 2)

            @pl.when(blast == 0)
            def _():
                write(KMAX - 1, buf0)

            @pl.when(blast == 1)
            def _():
                write(KMAX - 1, buf1)

    run(utab, uout)
    run(itab, iout)

    @pl.when(jnp.logical_and(w >= 1, w <= 4))
    def _():
        t = NGRP * GT + (w - 1)
        sl = pl.ds(pl.multiple_of(t * 128, 128), 128)
        pltpu.sync_copy(utab.at[:, sl], buf0.at[:, pl.ds(0, 128)])
        pltpu.sync_copy(buf0.at[:, pl.ds(0, 128)],
                        uout.at[pl.ds(t * 32, 32), :])
        pltpu.sync_copy(itab.at[:, sl], buf1.at[:, pl.ds(0, 128)])
        pltpu.sync_copy(buf1.at[:, pl.ds(0, 128)],
                        iout.at[pl.ds(t * 32, 32), :])

    @pl.when(w == 0)
    def _():
        pltpu.sync_copy(utail, buf0.at[:, pl.ds(0, 128)])
        pltpu.sync_copy(buf0.at[:, pl.ds(0, 128)],
                        uout.at[pl.ds(NTILE * 32, 32), :])
        pltpu.sync_copy(itail, buf1.at[:, pl.ds(0, 128)])
        pltpu.sync_copy(buf1.at[:, pl.ds(0, 128)],
                        iout.at[pl.ds(NTILE * 32, 32), :])


_retile = functools.partial(
    pl.kernel,
    mesh=plsc.VectorSubcoreMesh(core_axis_name="c", subcore_axis_name="s"),
    out_type=(jax.ShapeDtypeStruct((SROWS, 128), jnp.float32),
              jax.ShapeDtypeStruct((SROWS, 128), jnp.float32)),
    compiler_params=pltpu.CompilerParams(needs_layout_passes=False),
    scratch_types=[
        pltpu.VMEM((DIM, GT * 128), jnp.float32),
        pltpu.VMEM((DIM, GT * 128), jnp.float32),
        pltpu.SemaphoreType.DMA,
    ],
)(_retile_body)


def _score_body(uid_hbm, pid_hbm, nid_hbm, uflat, iflat, btab, out_hbm,
                uidx_v, pidx_v, nidx_v, ufi, pfi, nfi,
                urows, prows, nrows, bp, bn, outp, outn, sem):
    w = lax.axis_index("s") * 2 + lax.axis_index("c")
    base = w * PER_W

    pltpu.sync_copy(uid_hbm.at[pl.ds(base, PER_W)], uidx_v)
    pltpu.sync_copy(pid_hbm.at[pl.ds(base, PER_W)], pidx_v)
    pltpu.sync_copy(nid_hbm.at[pl.ds(base, PER_W)], nidx_v)

    bias_copies = []
    for c in range(NCHUNK):
        sl = pl.ds(c * CHUNK, CHUNK)
        bias_copies.append(pltpu.async_copy(btab.at[pidx_v.at[sl]], bp.at[sl], sem))
        bias_copies.append(pltpu.async_copy(btab.at[nidx_v.at[sl]], bn.at[sl], sem))
    def fill_base(c, idx_v, fi):
        for k in range(CHUNK // 16):
            r = idx_v[pl.ds(c * CHUNK + k * 16, 16)]
            b = lax.shift_left(lax.shift_right_logical(r, 7), 12) + \
                lax.bitwise_and(r, 127)
            for d in range(DIM):
                fi[pl.ds((c * DIM + d) * CHUNK + k * 16, 16)] = b + d * 128

    for c in range(NCHUNK):
        fill_base(c, uidx_v, ufi)
        fill_base(c, pidx_v, pfi)
        fill_base(c, nidx_v, nfi)
        for d in range(DIM):
            isl = pl.ds((c * DIM + d) * CHUNK, CHUNK)
            dst = pl.ds(d * PER_W + c * CHUNK, CHUNK)
            pltpu.async_copy(uflat.at[ufi.at[isl]], urows.at[dst], sem)
            pltpu.async_copy(iflat.at[pfi.at[isl]], prows.at[dst], sem)
            pltpu.async_copy(iflat.at[nfi.at[isl]], nrows.at[dst], sem)
    for c in bias_copies:
        c.wait()
    pltpu.make_async_copy(uflat.at[pl.ds(0, DIM * PER_W)], urows, sem).wait()
    pltpu.make_async_copy(uflat.at[pl.ds(0, DIM * PER_W)], prows, sem).wait()
    pltpu.make_async_copy(uflat.at[pl.ds(0, DIM * PER_W)], nrows, sem).wait()

    def group(g, _):
        gbase = g * 16
        accp = bp[pl.ds(gbase, 16)]
        accn = bn[pl.ds(gbase, 16)]
        for d in range(DIM):
            uv = urows[pl.ds(d * PER_W + gbase, 16)]
            pv = prows[pl.ds(d * PER_W + gbase, 16)]
            nv = nrows[pl.ds(d * PER_W + gbase, 16)]
            accp = accp + uv * pv
            accn = accn + uv * nv
        outp[pl.ds(gbase, 16)] = accp
        outn[pl.ds(gbase, 16)] = accn
        return 0

    lax.fori_loop(0, PER_W // 16, group, 0)

    pltpu.sync_copy(outp, out_hbm.at[pl.ds(base, PER_W)])
    pltpu.sync_copy(outn, out_hbm.at[pl.ds(base + BATCH, PER_W)])


_score = functools.partial(
    pl.kernel,
    mesh=plsc.VectorSubcoreMesh(core_axis_name="c", subcore_axis_name="s"),
    out_type=jax.ShapeDtypeStruct((2 * BATCH,), jnp.float32),
    compiler_params=pltpu.CompilerParams(needs_layout_passes=False),
    scratch_types=[
        pltpu.VMEM((PER_W,), jnp.int32),
        pltpu.VMEM((PER_W,), jnp.int32),
        pltpu.VMEM((PER_W,), jnp.int32),
        pltpu.VMEM((DIM * PER_W,), jnp.int32),
        pltpu.VMEM((DIM * PER_W,), jnp.int32),
        pltpu.VMEM((DIM * PER_W,), jnp.int32),
        pltpu.VMEM((DIM * PER_W,), jnp.float32),
        pltpu.VMEM((DIM * PER_W,), jnp.float32),
        pltpu.VMEM((DIM * PER_W,), jnp.float32),
        pltpu.VMEM((PER_W,), jnp.float32),
        pltpu.VMEM((PER_W,), jnp.float32),
        pltpu.VMEM((PER_W,), jnp.float32),
        pltpu.VMEM((PER_W,), jnp.float32),
        pltpu.SemaphoreType.DMA,
    ],
)(_score_body)


def kernel(batch_data, u_embedding, i_embedding, i_bias):
    idx = batch_data.astype(jnp.int32)
    tail = jnp.zeros((128, DIM), jnp.float32)
    utail = tail.at[:VOCAB - NTILE * 128].set(u_embedding[NTILE * 128:]).T
    itail = tail.at[:VOCAB - NTILE * 128].set(i_embedding[NTILE * 128:]).T
    ulin, ilin = _retile(u_embedding.T, i_embedding.T, utail, itail)
    logits = _score(idx[:, 0], idx[:, 1], idx[:, 2],
                    ulin.reshape(SROWS * 128), ilin.reshape(SROWS * 128),
                    i_bias.reshape(-1))
    labels = jnp.concatenate([
        jnp.ones((BATCH,), dtype=jnp.float32),
        jnp.zeros((BATCH,), dtype=jnp.float32),
    ])
    return (logits.reshape(2 * BATCH, 1), labels)
